# baseline (speedup 1.0000x reference)
import functools

import jax
import jax.numpy as jnp
from jax import lax
from jax.experimental import pallas as pl
from jax.experimental.pallas import tpu as pltpu
from jax.experimental.pallas import tpu_sc as plsc

_info = plsc.get_sparse_core_info()
_NC, _NS, _L = _info.num_cores, _info.num_subcores, _info.num_lanes
_NW = _NC * _NS

_B = 4
_S = 2048
_EMB = 1024
_P_PER_W = _S // _NW
_CP = 32
_NCHUNK = _P_PER_W // _CP
_VECS = _EMB // _L


def _body(x_hbm, tok_hbm, pos_hbm, out_hbm, idx_v, pos_v, tok_v, sem):
    wid = lax.axis_index("s") * _NC + lax.axis_index("c")
    pos0 = wid * _P_PER_W
    for pc in range(_NCHUNK):
        p = pos0 + pc * _CP
        pltpu.sync_copy(pos_hbm.at[pl.ds(p, _CP)], pos_v)
        for b in range(_B):
            pltpu.sync_copy(x_hbm.at[b, pl.ds(p, _CP)], idx_v)
            pltpu.async_copy(tok_hbm.at[idx_v], tok_v, sem).wait()

            def add_one(i, _):
                r = i // _VECS
                c = (i % _VECS) * _L
                plsc.addupdate(tok_v.at[r, pl.ds(c, _L)],
                               pos_v[r, pl.ds(c, _L)])
                return _

            lax.fori_loop(0, _CP * _VECS, add_one, None)
            pltpu.sync_copy(tok_v, out_hbm.at[b, pl.ds(p, _CP)])


@jax.jit
def _emb(x, token_table, pos_table):
    mesh = plsc.VectorSubcoreMesh(core_axis_name="c", subcore_axis_name="s")
    return pl.kernel(
        _body,
        out_type=jax.ShapeDtypeStruct((_B, _S, _EMB), jnp.float32),
        mesh=mesh,
        scratch_types=[
            pltpu.VMEM((_CP,), jnp.int32),
            pltpu.VMEM((_CP, _EMB), jnp.float32),
            pltpu.VMEM((_CP, _EMB), jnp.float32),
            pltpu.SemaphoreType.DMA,
        ],
    )(x, token_table, pos_table)


def kernel(x, token_table, pos_table):
    return _emb(x.astype(jnp.int32), token_table, pos_table)


# speedup vs baseline: 1.9554x; 1.9554x over previous
import jax
import jax.numpy as jnp
from jax import lax
from jax.experimental import pallas as pl
from jax.experimental.pallas import tpu as pltpu
from jax.experimental.pallas import tpu_sc as plsc

_info = plsc.get_sparse_core_info()
_NC, _NS, _L = _info.num_cores, _info.num_subcores, _info.num_lanes
_NW = _NC * _NS

_B = 4
_S = 2048
_EMB = 1024
_P_PER_W = _S // _NW
_CP = 16
_NCHUNK = _P_PER_W // _CP
_VECS = _EMB // _L
_NBUF = 3


def _body(x_hbm, tok_hbm, pos_hbm, out_hbm,
          idx0, idx1, idx2, tok0, tok1, tok2, pos_v,
          gsem0, gsem1, gsem2, osem0, osem1, osem2, psem):
    wid = lax.axis_index("s") * _NC + lax.axis_index("c")
    pos0 = wid * _P_PER_W
    idx = [idx0, idx1, idx2]
    tok = [tok0, tok1, tok2]
    gsem = [gsem0, gsem1, gsem2]
    osem = [osem0, osem1, osem2]

    pos_cp = pltpu.async_copy(pos_hbm.at[pl.ds(pos0, _P_PER_W)], pos_v, psem)

    units = [(pc, b) for pc in range(_NCHUNK) for b in range(_B)]
    g_desc = [None] * _NBUF
    o_desc = [None] * _NBUF

    def issue(u):
        slot = u % _NBUF
        pc, b = units[u]
        p = pos0 + pc * _CP
        pltpu.sync_copy(x_hbm.at[b, pl.ds(p, _CP)], idx[slot])
        g_desc[slot] = pltpu.async_copy(tok_hbm.at[idx[slot]], tok[slot],
                                        gsem[slot])

    issue(0)
    issue(1)
    pos_cp.wait()
    for u in range(len(units)):
        cur = u % _NBUF
        if u + 2 < len(units):
            nxt = (u + 2) % _NBUF
            if o_desc[nxt] is not None:
                o_desc[nxt].wait()
            issue(u + 2)
        g_desc[cur].wait()
        pc, b = units[u]
        tv = tok[cur]
        roff = pc * _CP

        @plsc.parallel_loop(0, _CP * _VECS, unroll=8)
        def _(i):
            r = i >> 6
            c = (i & (_VECS - 1)) * _L
            plsc.addupdate(tv.at[r, pl.ds(c, _L)],
                           pos_v[roff + r, pl.ds(c, _L)])

        p = pos0 + pc * _CP
        o_desc[cur] = pltpu.async_copy(tv, out_hbm.at[b, pl.ds(p, _CP)],
                                       osem[cur])
    for d in o_desc:
        d.wait()


@jax.jit
def _emb(x, token_table, pos_table):
    mesh = plsc.VectorSubcoreMesh(core_axis_name="c", subcore_axis_name="s")
    return pl.kernel(
        _body,
        out_type=jax.ShapeDtypeStruct((_B, _S, _EMB), jnp.float32),
        mesh=mesh,
        scratch_types=[
            pltpu.VMEM((_CP,), jnp.int32),
            pltpu.VMEM((_CP,), jnp.int32),
            pltpu.VMEM((_CP,), jnp.int32),
            pltpu.VMEM((_CP, _EMB), jnp.float32),
            pltpu.VMEM((_CP, _EMB), jnp.float32),
            pltpu.VMEM((_CP, _EMB), jnp.float32),
            pltpu.VMEM((_P_PER_W, _EMB), jnp.float32),
            pltpu.SemaphoreType.DMA,
            pltpu.SemaphoreType.DMA,
            pltpu.SemaphoreType.DMA,
            pltpu.SemaphoreType.DMA,
            pltpu.SemaphoreType.DMA,
            pltpu.SemaphoreType.DMA,
            pltpu.SemaphoreType.DMA,
        ],
    )(x, token_table, pos_table)


def kernel(x, token_table, pos_table):
    return _emb(x.astype(jnp.int32), token_table, pos_table)


# speedup vs baseline: 2.0658x; 1.0564x over previous
import jax
import jax.numpy as jnp
from jax import lax
from jax.experimental import pallas as pl
from jax.experimental.pallas import tpu as pltpu
from jax.experimental.pallas import tpu_sc as plsc

_info = plsc.get_sparse_core_info()
_NC, _NS, _L = _info.num_cores, _info.num_subcores, _info.num_lanes
_NW = _NC * _NS

_B = 4
_S = 2048
_EMB = 1024
_P_PER_W = _S // _NW
_CP = 8
_NCHUNK = _P_PER_W // _CP
_VECS = _EMB // _L
_NBUF = 4


def _body(x_hbm, tok_hbm, pos_hbm, out_hbm,
          idx_all, tok0, tok1, tok2, tok3, pos_v,
          isem, gsem0, gsem1, gsem2, gsem3, osem0, osem1, osem2, osem3,
          psem):
    wid = lax.axis_index("s") * _NC + lax.axis_index("c")
    pos0 = wid * _P_PER_W
    tok = [tok0, tok1, tok2, tok3]
    gsem = [gsem0, gsem1, gsem2, gsem3]
    osem = [osem0, osem1, osem2, osem3]

    icps = [pltpu.async_copy(x_hbm.at[b, pl.ds(pos0, _P_PER_W)],
                             idx_all.at[b], isem) for b in range(_B)]
    pos_cp = pltpu.async_copy(pos_hbm.at[pl.ds(pos0, _P_PER_W)], pos_v, psem)
    for c in icps:
        c.wait()

    units = [(pc, b) for pc in range(_NCHUNK) for b in range(_B)]
    nu = len(units)
    g_desc = [None] * _NBUF
    o_desc = [None] * _NBUF

    def issue(u):
        slot = u % _NBUF
        pc, b = units[u]
        g_desc[slot] = pltpu.async_copy(
            tok_hbm.at[idx_all.at[b, pl.ds(pc * _CP, _CP)]],
            tok[slot], gsem[slot])

    issue(0)
    issue(1)
    pos_cp.wait()
    for u in range(nu):
        cur = u % _NBUF
        g_desc[cur].wait()
        pc, b = units[u]
        tv = tok[cur]
        roff = pc * _CP

        @plsc.parallel_loop(0, _CP * _VECS, unroll=8)
        def _(i):
            r = i >> 6
            c = (i & (_VECS - 1)) * _L
            plsc.addupdate(tv.at[r, pl.ds(c, _L)],
                           pos_v[roff + r, pl.ds(c, _L)])

        p = pos0 + pc * _CP
        o_desc[cur] = pltpu.async_copy(tv, out_hbm.at[b, pl.ds(p, _CP)],
                                       osem[cur])
        if u + 2 < nu:
            nxt = (u + 2) % _NBUF
            if o_desc[nxt] is not None:
                o_desc[nxt].wait()
            issue(u + 2)
    o_desc[(nu - 2) % _NBUF].wait()
    o_desc[(nu - 1) % _NBUF].wait()


@jax.jit
def _emb(x, token_table, pos_table):
    mesh = plsc.VectorSubcoreMesh(core_axis_name="c", subcore_axis_name="s")
    return pl.kernel(
        _body,
        out_type=jax.ShapeDtypeStruct((_B, _S, _EMB), jnp.float32),
        mesh=mesh,
        scratch_types=[
            pltpu.VMEM((_B, _P_PER_W), jnp.int32),
            pltpu.VMEM((_CP, _EMB), jnp.float32),
            pltpu.VMEM((_CP, _EMB), jnp.float32),
            pltpu.VMEM((_CP, _EMB), jnp.float32),
            pltpu.VMEM((_CP, _EMB), jnp.float32),
            pltpu.VMEM((_P_PER_W, _EMB), jnp.float32),
            pltpu.SemaphoreType.DMA,
            pltpu.SemaphoreType.DMA,
            pltpu.SemaphoreType.DMA,
            pltpu.SemaphoreType.DMA,
            pltpu.SemaphoreType.DMA,
            pltpu.SemaphoreType.DMA,
            pltpu.SemaphoreType.DMA,
            pltpu.SemaphoreType.DMA,
            pltpu.SemaphoreType.DMA,
            pltpu.SemaphoreType.DMA,
        ],
    )(x, token_table, pos_table)


def kernel(x, token_table, pos_table):
    return _emb(x.astype(jnp.int32), token_table, pos_table)


# speedup vs baseline: 2.3419x; 1.1337x over previous
import jax
import jax.numpy as jnp
from jax import lax
from jax.experimental import pallas as pl
from jax.experimental.pallas import tpu as pltpu
from jax.experimental.pallas import tpu_sc as plsc

_info = plsc.get_sparse_core_info()
_NC, _NS, _L = _info.num_cores, _info.num_subcores, _info.num_lanes
_NW = _NC * _NS

_B = 4
_S = 2048
_EMB = 1024
_P_PER_W = _S // _NW
_CP = 8
_NCHUNK = _P_PER_W // _CP
_VECS = _EMB // _L
_NBUF = 6
_LA = 3


def _body(x_hbm, tok_hbm, pos_hbm, out_hbm,
          idx_all, tok0, tok1, tok2, tok3, tok4, tok5, pos_v,
          isem, gsem0, gsem1, gsem2, gsem3, gsem4, gsem5,
          osem0, osem1, osem2, osem3, osem4, osem5,
          psem):
    wid = lax.axis_index("s") * _NC + lax.axis_index("c")
    pos0 = wid * _P_PER_W
    tok = [tok0, tok1, tok2, tok3, tok4, tok5]
    gsem = [gsem0, gsem1, gsem2, gsem3, gsem4, gsem5]
    osem = [osem0, osem1, osem2, osem3, osem4, osem5]

    icps = [pltpu.async_copy(x_hbm.at[b, pl.ds(pos0, _P_PER_W)],
                             idx_all.at[b], isem) for b in range(_B)]
    pos_cp = pltpu.async_copy(pos_hbm.at[pl.ds(pos0, _P_PER_W)], pos_v, psem)
    for c in icps:
        c.wait()

    units = [(pc, b) for pc in range(_NCHUNK) for b in range(_B)]
    nu = len(units)
    g_desc = [None] * _NBUF
    o_desc = [None] * _NBUF

    def issue(u):
        slot = u % _NBUF
        pc, b = units[u]
        g_desc[slot] = pltpu.async_copy(
            tok_hbm.at[idx_all.at[b, pl.ds(pc * _CP, _CP)]],
            tok[slot], gsem[slot])

    for v in range(_LA):
        issue(v)
    pos_cp.wait()
    for u in range(nu):
        cur = u % _NBUF
        if u + _LA < nu:
            nxt = (u + _LA) % _NBUF
            if o_desc[nxt] is not None:
                o_desc[nxt].wait()
            issue(u + _LA)
        g_desc[cur].wait()
        pc, b = units[u]
        tv = tok[cur]
        roff = pc * _CP

        @plsc.parallel_loop(0, _CP * _VECS, unroll=8)
        def _(i):
            r = i >> 6
            c = (i & (_VECS - 1)) * _L
            plsc.addupdate(tv.at[r, pl.ds(c, _L)],
                           pos_v[roff + r, pl.ds(c, _L)])

        p = pos0 + pc * _CP
        o_desc[cur] = pltpu.async_copy(tv, out_hbm.at[b, pl.ds(p, _CP)],
                                       osem[cur])
    for v in range(nu - _NBUF, nu):
        o_desc[v % _NBUF].wait()


@jax.jit
def _emb(x, token_table, pos_table):
    mesh = plsc.VectorSubcoreMesh(core_axis_name="c", subcore_axis_name="s")
    return pl.kernel(
        _body,
        out_type=jax.ShapeDtypeStruct((_B, _S, _EMB), jnp.float32),
        mesh=mesh,
        scratch_types=(
            [pltpu.VMEM((_B, _P_PER_W), jnp.int32)]
            + [pltpu.VMEM((_CP, _EMB), jnp.float32) for _ in range(_NBUF)]
            + [pltpu.VMEM((_P_PER_W, _EMB), jnp.float32)]
            + [pltpu.SemaphoreType.DMA for _ in range(2 * _NBUF + 2)]
        ),
    )(x, token_table, pos_table)


def kernel(x, token_table, pos_table):
    return _emb(x.astype(jnp.int32), token_table, pos_table)
